# Initial kernel scaffold; baseline (speedup 1.0000x reference)
import jax
import jax.numpy as jnp
from jax.experimental import pallas as pl


def kernel(center_words, context_words, neg_context_words, input_embeddings, output_embeddings):
    raise NotImplementedError("write your pallas kernel here")


# speedup vs baseline: 5.2235x; 5.2235x over previous
import functools

import jax
import jax.numpy as jnp
from jax import lax
from jax.experimental import pallas as pl
from jax.experimental.pallas import tpu as pltpu
from jax.experimental.pallas import tpu_sc as plsc

B = 16384
D = 64
NEG = 20
L = 16
NVR = D // L

NC = 2
NS = 16
NW = NC * NS
BPW = B // NW
C = 64
NCH = BPW // C

_mesh = plsc.VectorSubcoreMesh(core_axis_name="c", subcore_axis_name="s")


@functools.partial(
    pl.kernel,
    mesh=_mesh,
    compiler_params=pltpu.CompilerParams(use_tc_tiling_on_sc=False),
    out_type=(
        jax.ShapeDtypeStruct((B, L), jnp.float32),
        jax.ShapeDtypeStruct((B, L), jnp.float32),
    ),
    scratch_types=[
        pltpu.VMEM((C,), jnp.int32),
        pltpu.VMEM((C,), jnp.int32),
        pltpu.VMEM((NEG * C,), jnp.int32),
        pltpu.VMEM((C, D), jnp.float32),
        pltpu.VMEM((C, D), jnp.float32),
        pltpu.VMEM((NEG * C, D), jnp.float32),
        pltpu.VMEM((C, L), jnp.float32),
        pltpu.VMEM((C, L), jnp.float32),
        pltpu.SemaphoreType.DMA,
        pltpu.SemaphoreType.DMA,
        pltpu.SemaphoreType.DMA,
    ],
)
def _sc_partials(cidx_hbm, kidx_hbm, nidx_hbm, iemb_hbm, oemb_hbm,
                 posp_hbm, negp_hbm,
                 cidx_v, kidx_v, nidx_v, crow_v, krow_v, nrow_v,
                 posp_v, negp_v, sem_c, sem_k, sem_n):
    wid = lax.axis_index("s") * NC + lax.axis_index("c")
    for ch in range(NCH):
        base = wid * BPW + ch * C
        pltpu.sync_copy(cidx_hbm.at[pl.ds(base, C)], cidx_v)
        pltpu.sync_copy(kidx_hbm.at[pl.ds(base, C)], kidx_v)
        pltpu.sync_copy(nidx_hbm.at[wid, ch], nidx_v)
        cp_c = pltpu.async_copy(iemb_hbm.at[cidx_v], crow_v, sem_c)
        cp_k = pltpu.async_copy(oemb_hbm.at[kidx_v], krow_v, sem_k)
        cp_n = pltpu.async_copy(oemb_hbm.at[nidx_v], nrow_v, sem_n)
        cp_c.wait()
        cp_k.wait()
        cp_n.wait()

        def row_body(r, carry):
            c = [crow_v[r, pl.ds(L * v, L)] for v in range(NVR)]
            k = [krow_v[r, pl.ds(L * v, L)] for v in range(NVR)]
            pp = c[0] * k[0] + c[1] * k[1] + c[2] * k[2] + c[3] * k[3]
            posp_v[r, :] = pp
            s = [nrow_v[r, pl.ds(L * v, L)] for v in range(NVR)]
            for n in range(1, NEG):
                for v in range(NVR):
                    s[v] = s[v] + nrow_v[n * C + r, pl.ds(L * v, L)]
            np_ = c[0] * s[0] + c[1] * s[1] + c[2] * s[2] + c[3] * s[3]
            negp_v[r, :] = np_
            return carry

        lax.fori_loop(0, C, row_body, 0)
        pltpu.sync_copy(posp_v, posp_hbm.at[pl.ds(base, C)])
        pltpu.sync_copy(negp_v, negp_hbm.at[pl.ds(base, C)])


def _tc_loss_body(pp_ref, np_ref, out_ref):
    sp = jnp.sum(pp_ref[...], axis=1, keepdims=True)
    sn = jnp.sum(np_ref[...], axis=1, keepdims=True)

    def logsig(x):
        return jnp.minimum(x, 0.0) - jnp.log(1.0 + jnp.exp(-jnp.abs(x)))

    tot = jnp.sum(logsig(sp) + logsig(-sn))
    out_ref[...] = jnp.full((1, 1), -(tot / B), dtype=jnp.float32)


_tc_loss = pl.pallas_call(
    _tc_loss_body,
    out_shape=jax.ShapeDtypeStruct((1, 1), jnp.float32),
)


def kernel(center_words, context_words, neg_context_words,
           input_embeddings, output_embeddings):
    cidx = center_words.astype(jnp.int32)
    kidx = context_words.astype(jnp.int32)
    nidx = (neg_context_words.astype(jnp.int32)
            .reshape(NW, NCH, C, NEG)
            .transpose(0, 1, 3, 2)
            .reshape(NW, NCH, NEG * C))
    posp, negp = _sc_partials(cidx, kidx, nidx,
                              input_embeddings, output_embeddings)
    return _tc_loss(posp, negp)[0, 0]
